# baseline (speedup 1.0000x reference)
import functools

import jax
import jax.numpy as jnp
from jax import lax
from jax.experimental import pallas as pl
from jax.experimental.pallas import tpu as pltpu
from jax.experimental.pallas import tpu_sc as plsc

_B, _T = 512, 50
_K = 1000
_D = 16
_N = _B * _T


def _mlp_body(fm_ref, wh_ref, bh_ref, wo_ref, bo_ref, o_ref):
    h = jnp.tanh(jnp.dot(fm_ref[...], wh_ref[...],
                         preferred_element_type=jnp.float32) + bh_ref[...])
    o_ref[...] = jnp.dot(h, wo_ref[...],
                         preferred_element_type=jnp.float32) + bo_ref[...]


def _lae(a, b):
    m = jnp.maximum(a, b)
    return m + jnp.log1p(jnp.exp(jnp.minimum(a, b) - m))


def _recur_body(g_ref, o0_ref, o1_ref, kc_ref, corr_ref, out_ref, ah_ref,
                pre_ref, lap_ref):
    def gs(c):
        return g_ref[c * _T:(c + 1) * _T]

    G0, G1, G2, G3, G4 = gs(0), gs(1), gs(2), gs(3), gs(4)
    G5, G6, G7, G8, G9 = gs(5), gs(6), gs(7), gs(8), gs(9)
    o0s = o0_ref[...].T
    o1s = o1_ref[...].T
    ol00 = G4 + o0s
    ol01 = G5 - o0s
    ol10 = G6 + o1s
    ol11 = G7 - o1s
    d0 = _lae(ol00, ol01)
    d1 = _lae(ol10, ol11)
    n0 = _lae(G0, G2)
    n1 = _lae(G1, G3)
    ni = _lae(G8, G9)
    y0s = corr_ref[...] == 0
    pre_ref[0 * _T:1 * _T] = G0 - n0
    pre_ref[1 * _T:2 * _T] = G1 - n1
    pre_ref[2 * _T:3 * _T] = G2 - n0
    pre_ref[3 * _T:4 * _T] = G3 - n1
    pre_ref[4 * _T:5 * _T] = jnp.where(y0s, ol00, ol01)
    pre_ref[5 * _T:6 * _T] = jnp.where(y0s, ol10, ol11)
    pre_ref[6 * _T:7 * _T] = G8 - ni - d0
    pre_ref[7 * _T:8 * _T] = G9 - ni - d1
    pre_ref[8 * _T:9 * _T] = d0
    pre_ref[9 * _T:10 * _T] = d1

    kc_all = kc_ref[...]
    iota_t = lax.broadcasted_iota(jnp.int32, (_T, _B), 0)

    def step(t):
        t8 = min(-(-max(t, 1) // 8) * 8, _T)

        def pr(c):
            return pre_ref[pl.ds(c * _T + t, 1)]

        if t == 0:
            lap0, lap1 = pr(6), pr(7)
        else:
            kct = kc_ref[pl.ds(t, 1)]
            eq = (kc_all[0:t8] == kct) & (iota_t[0:t8] < t)
            previ = jnp.max(jnp.where(eq, iota_t[0:t8], -1), axis=0,
                            keepdims=True)
            sel = iota_t[0:t8] == previ
            a0g = jnp.sum(jnp.where(sel, ah_ref[0:t8, :], 0.0), axis=0,
                          keepdims=True)
            a1g = jnp.sum(jnp.where(sel, ah_ref[_T:_T + t8, :], 0.0), axis=0,
                          keepdims=True)
            has = previ >= 0
            lap0 = jnp.where(has, a0g - pr(8), pr(6))
            lap1 = jnp.where(has, a1g - pr(9), pr(7))
        lap_ref[pl.ds(t, 1)] = lap0
        lap_ref[pl.ds(_T + t, 1)] = lap1
        c0 = pr(4) + lap0
        c1 = pr(5) + lap1
        ah_ref[pl.ds(t, 1)] = _lae(c0 + pr(0), c1 + pr(1))
        ah_ref[pl.ds(_T + t, 1)] = _lae(c0 + pr(2), c1 + pr(3))

    for t in range(_T):
        step(t)

    LA0 = lap_ref[0:_T]
    LA1 = lap_ref[_T:2 * _T]
    P0 = _lae(ol00 + LA0, ol10 + LA1)
    P1 = _lae(ol01 + LA0, ol11 + LA1)
    NZ = _lae(P0, P1)
    out_ref[0:_T] = P0 - NZ
    out_ref[_T:2 * _T] = P1 - NZ


def _sc_gather(table, idx):
    info = plsc.get_sparse_core_info()
    nc, ns = info.num_cores, info.num_subcores
    nw = nc * ns
    n_per_w = _N // nw
    mesh = plsc.VectorSubcoreMesh(core_axis_name="c", subcore_axis_name="s")

    @functools.partial(
        pl.kernel, mesh=mesh,
        compiler_params=pltpu.CompilerParams(use_tc_tiling_on_sc=False),
        out_type=jax.ShapeDtypeStruct((_N, _D), jnp.float32),
        scratch_types=[
            pltpu.VMEM((n_per_w,), jnp.int32),
            pltpu.VMEM((n_per_w, _D), jnp.float32),
            pltpu.SemaphoreType.DMA,
        ],
    )
    def gather_k(table_hbm, idx_hbm, out_hbm, idx_v, rows_v, sem):
        wid = lax.axis_index("s") * nc + lax.axis_index("c")
        base = wid * n_per_w
        pltpu.sync_copy(idx_hbm.at[pl.ds(base, n_per_w)], idx_v)
        pltpu.async_copy(table_hbm.at[idx_v], rows_v, sem).wait()
        pltpu.sync_copy(rows_v, out_hbm.at[pl.ds(base, n_per_w)])

    return gather_k(table, idx)


def kernel(corr, kc, FM, W_h, b_h, W_o, b_o, trans_logits, obs_logits, init_logits):
    mb = 6400
    o = pl.pallas_call(
        _mlp_body,
        grid=(_N // mb,),
        in_specs=[
            pl.BlockSpec((mb, 128), lambda i: (i, 0)),
            pl.BlockSpec((128, 64), lambda i: (0, 0)),
            pl.BlockSpec((1, 64), lambda i: (0, 0)),
            pl.BlockSpec((64, 2), lambda i: (0, 0)),
            pl.BlockSpec((1, 2), lambda i: (0, 0)),
        ],
        out_specs=pl.BlockSpec((mb, 2), lambda i: (i, 0)),
        out_shape=jax.ShapeDtypeStruct((_N, 2), jnp.float32),
    )(FM.reshape(_N, 128), W_h, b_h.reshape(1, 64), W_o, b_o.reshape(1, 2))
    o3 = o.reshape(_B, _T, 2)
    o0_raw = o3[:, :, 0]
    o1_raw = o3[:, :, 1]

    table = jnp.concatenate([
        trans_logits.reshape(_K, 4),
        obs_logits.reshape(_K, 4),
        init_logits,
        jnp.zeros((_K, _D - 10), jnp.float32),
    ], axis=1)
    kc_t = kc.T.astype(jnp.int32)
    idx = kc_t.reshape(-1)
    gathered = _sc_gather(table, idx)
    g_l = gathered.reshape(_T, _B, _D).transpose(2, 0, 1).reshape(_D * _T, _B)

    res = pl.pallas_call(
        _recur_body,
        out_shape=jax.ShapeDtypeStruct((2 * _T, _B), jnp.float32),
        scratch_shapes=[pltpu.VMEM((2 * _T, _B), jnp.float32),
                        pltpu.VMEM((10 * _T, _B), jnp.float32),
                        pltpu.VMEM((2 * _T, _B), jnp.float32)],
    )(g_l, o0_raw, o1_raw, kc_t, corr.T.astype(jnp.int32))

    return res.reshape(2, _T, _B).transpose(2, 1, 0)


# speedup vs baseline: 1.0976x; 1.0976x over previous
import functools

import jax
import jax.numpy as jnp
from jax import lax
from jax.experimental import pallas as pl
from jax.experimental.pallas import tpu as pltpu
from jax.experimental.pallas import tpu_sc as plsc

_B, _T = 512, 50
_K = 1000
_D = 16
_N = _B * _T


def _mlp_body(fm_ref, wh_ref, bh_ref, wo_ref, bo_ref, o_ref):
    h = jnp.tanh(jnp.dot(fm_ref[...], wh_ref[...],
                         preferred_element_type=jnp.float32) + bh_ref[...])
    o = jnp.dot(h, wo_ref[...],
                preferred_element_type=jnp.float32) + bo_ref[...]
    o_ref[...] = o.T


def _lae(a, b):
    m = jnp.maximum(a, b)
    return m + jnp.log1p(jnp.exp(jnp.minimum(a, b) - m))


def _recur_body(g_ref, o0_ref, o1_ref, kc_ref, corr_ref, out_ref, ah_ref,
                pre_ref, lap_ref):
    def gs(c):
        return g_ref[c * _T:(c + 1) * _T]

    G0, G1, G2, G3, G4 = gs(0), gs(1), gs(2), gs(3), gs(4)
    G5, G6, G7, G8, G9 = gs(5), gs(6), gs(7), gs(8), gs(9)
    o0s = o0_ref[...].T
    o1s = o1_ref[...].T
    ol00 = G4 + o0s
    ol01 = G5 - o0s
    ol10 = G6 + o1s
    ol11 = G7 - o1s
    d0 = _lae(ol00, ol01)
    d1 = _lae(ol10, ol11)
    n0 = _lae(G0, G2)
    n1 = _lae(G1, G3)
    ni = _lae(G8, G9)
    y0s = corr_ref[...] == 0
    pre_ref[0 * _T:1 * _T] = G0 - n0
    pre_ref[1 * _T:2 * _T] = G1 - n1
    pre_ref[2 * _T:3 * _T] = G2 - n0
    pre_ref[3 * _T:4 * _T] = G3 - n1
    pre_ref[4 * _T:5 * _T] = jnp.where(y0s, ol00, ol01)
    pre_ref[5 * _T:6 * _T] = jnp.where(y0s, ol10, ol11)
    pre_ref[6 * _T:7 * _T] = G8 - ni - d0
    pre_ref[7 * _T:8 * _T] = G9 - ni - d1
    pre_ref[8 * _T:9 * _T] = d0
    pre_ref[9 * _T:10 * _T] = d1

    kc_all = kc_ref[...]
    iota_t = lax.broadcasted_iota(jnp.int32, (_T, _B), 0)

    def step(t):
        t8 = min(-(-max(t, 1) // 8) * 8, _T)

        def pr(c):
            return pre_ref[pl.ds(c * _T + t, 1)]

        if t == 0:
            lap0, lap1 = pr(6), pr(7)
        else:
            kct = kc_ref[pl.ds(t, 1)]
            eq = (kc_all[0:t8] == kct) & (iota_t[0:t8] < t)
            previ = jnp.max(jnp.where(eq, iota_t[0:t8], -1), axis=0,
                            keepdims=True)
            sel = iota_t[0:t8] == previ
            a0g = jnp.sum(jnp.where(sel, ah_ref[0:t8, :], 0.0), axis=0,
                          keepdims=True)
            a1g = jnp.sum(jnp.where(sel, ah_ref[_T:_T + t8, :], 0.0), axis=0,
                          keepdims=True)
            has = previ >= 0
            lap0 = jnp.where(has, a0g - pr(8), pr(6))
            lap1 = jnp.where(has, a1g - pr(9), pr(7))
        lap_ref[pl.ds(t, 1)] = lap0
        lap_ref[pl.ds(_T + t, 1)] = lap1
        c0 = pr(4) + lap0
        c1 = pr(5) + lap1
        ah_ref[pl.ds(t, 1)] = _lae(c0 + pr(0), c1 + pr(1))
        ah_ref[pl.ds(_T + t, 1)] = _lae(c0 + pr(2), c1 + pr(3))

    for t in range(_T):
        step(t)

    LA0 = lap_ref[0:_T]
    LA1 = lap_ref[_T:2 * _T]
    P0 = _lae(ol00 + LA0, ol10 + LA1)
    P1 = _lae(ol01 + LA0, ol11 + LA1)
    NZ = _lae(P0, P1)
    out_ref[0:_T] = P0 - NZ
    out_ref[_T:2 * _T] = P1 - NZ


def _sc_gather(table, idx):
    info = plsc.get_sparse_core_info()
    nc, ns = info.num_cores, info.num_subcores
    nw = nc * ns
    n_per_w = _N // nw
    mesh = plsc.VectorSubcoreMesh(core_axis_name="c", subcore_axis_name="s")

    @functools.partial(
        pl.kernel, mesh=mesh,
        compiler_params=pltpu.CompilerParams(use_tc_tiling_on_sc=False),
        out_type=jax.ShapeDtypeStruct((_N, _D), jnp.float32),
        scratch_types=[
            pltpu.VMEM((n_per_w,), jnp.int32),
            pltpu.VMEM((n_per_w, _D), jnp.float32),
            pltpu.SemaphoreType.DMA,
        ],
    )
    def gather_k(table_hbm, idx_hbm, out_hbm, idx_v, rows_v, sem):
        wid = lax.axis_index("s") * nc + lax.axis_index("c")
        base = wid * n_per_w
        pltpu.sync_copy(idx_hbm.at[pl.ds(base, n_per_w)], idx_v)
        pltpu.async_copy(table_hbm.at[idx_v], rows_v, sem).wait()
        pltpu.sync_copy(rows_v, out_hbm.at[pl.ds(base, n_per_w)])

    return gather_k(table, idx)


def kernel(corr, kc, FM, W_h, b_h, W_o, b_o, trans_logits, obs_logits, init_logits):
    mb = 6400
    o = pl.pallas_call(
        _mlp_body,
        grid=(_N // mb,),
        in_specs=[
            pl.BlockSpec((mb, 128), lambda i: (i, 0)),
            pl.BlockSpec((128, 64), lambda i: (0, 0)),
            pl.BlockSpec((1, 64), lambda i: (0, 0)),
            pl.BlockSpec((64, 2), lambda i: (0, 0)),
            pl.BlockSpec((1, 2), lambda i: (0, 0)),
        ],
        out_specs=pl.BlockSpec((2, mb), lambda i: (0, i)),
        out_shape=jax.ShapeDtypeStruct((2, _N), jnp.float32),
    )(FM.reshape(_N, 128), W_h, b_h.reshape(1, 64), W_o, b_o.reshape(1, 2))
    o0_raw = o[0].reshape(_B, _T)
    o1_raw = o[1].reshape(_B, _T)

    table = jnp.concatenate([
        trans_logits.reshape(_K, 4),
        obs_logits.reshape(_K, 4),
        init_logits,
        jnp.zeros((_K, _D - 10), jnp.float32),
    ], axis=1)
    kc_t = kc.T.astype(jnp.int32)
    idx = kc_t.reshape(-1)
    gathered = _sc_gather(table, idx)
    g_l = gathered.reshape(_T, _B, _D).transpose(2, 0, 1).reshape(_D * _T, _B)

    res = pl.pallas_call(
        _recur_body,
        out_shape=jax.ShapeDtypeStruct((2 * _T, _B), jnp.float32),
        scratch_shapes=[pltpu.VMEM((2 * _T, _B), jnp.float32),
                        pltpu.VMEM((10 * _T, _B), jnp.float32),
                        pltpu.VMEM((2 * _T, _B), jnp.float32)],
    )(g_l, o0_raw, o1_raw, kc_t, corr.T.astype(jnp.int32))

    return res.reshape(2, _T, _B).transpose(2, 1, 0)
